# baseline (speedup 1.0000x reference)
import functools

import jax
import jax.numpy as jnp
from jax import lax
from jax.experimental import pallas as pl
from jax.experimental.pallas import tpu as pltpu
from jax.experimental.pallas import tpu_sc as plsc

_N = 100000
_D = 128
_K = 16
_L = 16
_G = _D // _L

_NC = 2
_NS = 16
_NW = _NC * _NS
_WROWS = _N // _NW
_CH_ROWS = 125
_CH_F = _CH_ROWS * _D
_NCHUNK = _WROWS // _CH_ROWS
_RU = 5


def _sc_body(x_hbm, out_hbm, buf0, buf1, part, sem0, sem1):
    wid = lax.axis_index("s") * _NC + lax.axis_index("c")
    base = wid * (_WROWS * _D)

    def issue(buf, sem, chunk):
        return pltpu.async_copy(
            x_hbm.at[pl.ds(base + chunk * _CH_F, _CH_F)], buf, sem)

    def wait(buf, sem):
        pltpu.make_async_copy(x_hbm.at[pl.ds(0, _CH_F)], buf, sem).wait()

    def accum(buf, accs):
        def rows_body(r, accs):
            accs = list(accs)
            off = r * (_RU * _D)
            for rr in range(_RU):
                for g in range(_G):
                    v = buf[pl.ds(off + rr * _D + g * _L, _L)]
                    accs[g] = accs[g] + v
                    accs[_G + g] = accs[_G + g] + v * v
            return tuple(accs)

        return lax.fori_loop(0, _CH_ROWS // _RU, rows_body, accs,
                             unroll=False)

    issue(buf0, sem0, 0)
    issue(buf1, sem1, 1)
    zero = jnp.zeros((_L,), jnp.float32)
    accs = tuple(zero for _ in range(2 * _G))

    def pair_body(jj, accs):
        c = jj * 2
        wait(buf0, sem0)
        accs = accum(buf0, accs)

        @pl.when(c + 2 < _NCHUNK)
        def _():
            issue(buf0, sem0, c + 2)

        wait(buf1, sem1)
        accs = accum(buf1, accs)

        @pl.when(c + 3 < _NCHUNK)
        def _():
            issue(buf1, sem1, c + 3)

        return accs

    accs = lax.fori_loop(0, (_NCHUNK - 1) // 2, pair_body, accs,
                         unroll=False)
    wait(buf0, sem0)
    accs = accum(buf0, accs)

    for g in range(_G):
        part[0, pl.ds(g * _L, _L)] = accs[g]
        part[1, pl.ds(g * _L, _L)] = accs[_G + g]
    pltpu.sync_copy(part, out_hbm.at[pl.ds(2 * wid, 2)])


@functools.partial(
    pl.kernel,
    out_type=jax.ShapeDtypeStruct((2 * _NW, _D), jnp.float32),
    mesh=plsc.VectorSubcoreMesh(core_axis_name="c", subcore_axis_name="s"),
    scratch_types=[
        pltpu.VMEM((_CH_F,), jnp.float32),
        pltpu.VMEM((_CH_F,), jnp.float32),
        pltpu.VMEM((2, _D), jnp.float32),
        pltpu.SemaphoreType.DMA,
        pltpu.SemaphoreType.DMA,
    ],
)
def _sc_partials(x_hbm, out_hbm, buf0, buf1, part, sem0, sem1):
    _sc_body(x_hbm, out_hbm, buf0, buf1, part, sem0, sem1)


def _finish_body(gid_ref, sel_ref, parts_ref, out_ref):
    sub = gid_ref[0] % 8
    rows = jax.lax.broadcasted_iota(jnp.int32, (8, _D), 0)
    sel = jnp.sum(jnp.where(rows == sub, sel_ref[...], 0.0), axis=0,
                  keepdims=True)
    p = parts_ref[...].reshape(_NW, 2, _D)
    m = _NW
    while m > 1:
        half = m // 2
        p = p[:half] + p[half:]
        m = half
    s1 = p[0, 0:1, :]
    s2 = p[0, 1:2, :]
    d2 = s2 - 2.0 * sel * s1 + jnp.float32(_N) * sel * sel
    dist = jnp.sqrt(jnp.maximum(d2, 0.0))
    lane = jax.lax.broadcasted_iota(jnp.int32, (1, _D), 1)
    total = jnp.float32(0.0)
    work = dist
    for _ in range(_K):
        mn = jnp.min(work)
        total = total + mn
        hit = work == mn
        first = jnp.min(jnp.where(hit, lane, _D))
        work = jnp.where(lane == first, jnp.float32(jnp.inf), work)
    out_ref[0, 0] = total / _K


@jax.jit
def _run(gid, x):
    parts = _sc_partials(x.reshape(-1))
    grid_spec = pltpu.PrefetchScalarGridSpec(
        num_scalar_prefetch=1,
        grid=(1,),
        in_specs=[
            pl.BlockSpec((8, _D), lambda i, g: (g[0] // 8, 0)),
            pl.BlockSpec((2 * _NW, _D), lambda i, g: (0, 0)),
        ],
        out_specs=pl.BlockSpec(memory_space=pltpu.SMEM),
    )
    out = pl.pallas_call(
        _finish_body,
        grid_spec=grid_spec,
        out_shape=jax.ShapeDtypeStruct((1, 1), jnp.float32),
    )(gid, x, parts)
    return out[0, 0]


def kernel(group_id, all_latents):
    gid = jnp.asarray(group_id, jnp.int32).reshape(1)
    return _run(gid, all_latents)


# speedup vs baseline: 1.2205x; 1.2205x over previous
import functools

import jax
import jax.numpy as jnp
from jax import lax
from jax.experimental import pallas as pl
from jax.experimental.pallas import tpu as pltpu
from jax.experimental.pallas import tpu_sc as plsc

_N = 100000
_D = 128
_K = 16

_TC_ROWS = 52000
_SC_ROWS = _N - _TC_ROWS

_BLK = 13000
_GRID = _TC_ROWS // _BLK
_U = 25
_C = _BLK // (8 * _U)

_L = 16
_G = _D // _L
_NC = 2
_NS = 16
_NW = _NC * _NS
_WROWS = _SC_ROWS // _NW
_CH_ROWS = 125
_CH_F = _CH_ROWS * _D
_NCHUNK = _WROWS // _CH_ROWS
assert _WROWS % _CH_ROWS == 0 and _TC_ROWS % _BLK == 0


def _sc_body(x_hbm, out_hbm, buf0, buf1, part, sem0, sem1):
    wid = lax.axis_index("s") * _NC + lax.axis_index("c")
    base = (_TC_ROWS + wid * _WROWS) * _D

    def issue(buf, sem, chunk):
        return pltpu.async_copy(
            x_hbm.at[pl.ds(base + chunk * _CH_F, _CH_F)], buf, sem)

    def wait(buf, sem):
        pltpu.make_async_copy(x_hbm.at[pl.ds(0, _CH_F)], buf, sem).wait()

    def accum(buf, accs):
        def rows_body(r, accs):
            accs = list(accs)
            off = r * (5 * _D)
            for rr in range(5):
                for g in range(_G):
                    v = buf[pl.ds(off + rr * _D + g * _L, _L)]
                    accs[g] = accs[g] + v
                    accs[_G + g] = accs[_G + g] + v * v
            return tuple(accs)

        return lax.fori_loop(0, _CH_ROWS // 5, rows_body, accs,
                             unroll=False)

    issue(buf0, sem0, 0)
    issue(buf1, sem1, 1)
    zero = jnp.zeros((_L,), jnp.float32)
    accs = tuple(zero for _ in range(2 * _G))

    def pair_body(jj, accs):
        c = jj * 2
        wait(buf0, sem0)
        accs = accum(buf0, accs)

        @pl.when(c + 2 < _NCHUNK)
        def _():
            issue(buf0, sem0, c + 2)

        wait(buf1, sem1)
        accs = accum(buf1, accs)

        @pl.when(c + 3 < _NCHUNK)
        def _():
            issue(buf1, sem1, c + 3)

        return accs

    accs = lax.fori_loop(0, _NCHUNK // 2, pair_body, accs, unroll=False)
    if _NCHUNK % 2:
        wait(buf0, sem0)
        accs = accum(buf0, accs)

    for g in range(_G):
        part[0, pl.ds(g * _L, _L)] = accs[g]
        part[1, pl.ds(g * _L, _L)] = accs[_G + g]
    pltpu.sync_copy(part, out_hbm.at[pl.ds(2 * wid, 2)])


@functools.partial(
    pl.kernel,
    out_type=jax.ShapeDtypeStruct((2 * _NW, _D), jnp.float32),
    mesh=plsc.VectorSubcoreMesh(core_axis_name="c", subcore_axis_name="s"),
    scratch_types=[
        pltpu.VMEM((_CH_F,), jnp.float32),
        pltpu.VMEM((_CH_F,), jnp.float32),
        pltpu.VMEM((2, _D), jnp.float32),
        pltpu.SemaphoreType.DMA,
        pltpu.SemaphoreType.DMA,
    ],
)
def _sc_partials(x_hbm, out_hbm, buf0, buf1, part, sem0, sem1):
    _sc_body(x_hbm, out_hbm, buf0, buf1, part, sem0, sem1)


def _sel_row(gid_ref, sel_ref):
    sub = gid_ref[0] % 8
    rows = jax.lax.broadcasted_iota(jnp.int32, (8, _D), 0)
    return jnp.sum(jnp.where(rows == sub, sel_ref[...], 0.0), axis=0,
                   keepdims=True)


def _tc_body(gid_ref, sel_ref, x_ref, out_ref):
    i = pl.program_id(0)

    @pl.when(i == 0)
    def _init():
        out_ref[...] = jnp.zeros_like(out_ref)

    sel = _sel_row(gid_ref, sel_ref)
    y = x_ref[...].reshape(_U, _C * 8, _D)
    acc = None
    for c in range(_C):
        d = y[:, c * 8:(c + 1) * 8, :] - sel
        s = d * d
        acc = s if acc is None else acc + s
    m = _U
    while m > 1:
        half = m // 2
        rest = acc[2 * half:]
        acc = acc[:half] + acc[half:2 * half]
        if rest.shape[0]:
            acc = jnp.concatenate([acc, rest], axis=0)
        m = acc.shape[0]
    out_ref[...] += acc[0]


def _finish_body(gid_ref, sel_ref, tc_ref, parts_ref, out_ref):
    sel = _sel_row(gid_ref, sel_ref)
    p = parts_ref[...].reshape(_NW, 2, _D)
    m = _NW
    while m > 1:
        half = m // 2
        p = p[:half] + p[half:]
        m = half
    s1 = p[0, 0:1, :]
    s2 = p[0, 1:2, :]
    d2_sc = s2 - 2.0 * sel * s1 + jnp.float32(_SC_ROWS) * sel * sel
    d2 = d2_sc + jnp.sum(tc_ref[...], axis=0, keepdims=True)
    dist = jnp.sqrt(jnp.maximum(d2, 0.0))
    lane = jax.lax.broadcasted_iota(jnp.int32, (1, _D), 1)
    total = jnp.float32(0.0)
    work = dist
    for _ in range(_K):
        mn = jnp.min(work)
        total = total + mn
        hit = work == mn
        first = jnp.min(jnp.where(hit, lane, _D))
        work = jnp.where(lane == first, jnp.float32(jnp.inf), work)
    out_ref[0, 0] = total / _K


@jax.jit
def _run(gid, x):
    parts = _sc_partials(x.reshape(-1))
    tc_spec = pltpu.PrefetchScalarGridSpec(
        num_scalar_prefetch=1,
        grid=(_GRID,),
        in_specs=[
            pl.BlockSpec((8, _D), lambda i, g: (g[0] // 8, 0)),
            pl.BlockSpec((_BLK, _D), lambda i, g: (i, 0)),
        ],
        out_specs=pl.BlockSpec((8, _D), lambda i, g: (0, 0)),
    )
    tc_part = pl.pallas_call(
        _tc_body,
        grid_spec=tc_spec,
        out_shape=jax.ShapeDtypeStruct((8, _D), jnp.float32),
        compiler_params=pltpu.CompilerParams(
            dimension_semantics=("arbitrary",)),
    )(gid, x, x)
    fin_spec = pltpu.PrefetchScalarGridSpec(
        num_scalar_prefetch=1,
        grid=(1,),
        in_specs=[
            pl.BlockSpec((8, _D), lambda i, g: (g[0] // 8, 0)),
            pl.BlockSpec((8, _D), lambda i, g: (0, 0)),
            pl.BlockSpec((2 * _NW, _D), lambda i, g: (0, 0)),
        ],
        out_specs=pl.BlockSpec(memory_space=pltpu.SMEM),
    )
    out = pl.pallas_call(
        _finish_body,
        grid_spec=fin_spec,
        out_shape=jax.ShapeDtypeStruct((1, 1), jnp.float32),
    )(gid, x, tc_part, parts)
    return out[0, 0]


def kernel(group_id, all_latents):
    gid = jnp.asarray(group_id, jnp.int32).reshape(1)
    return _run(gid, all_latents)


# speedup vs baseline: 2.2429x; 1.8376x over previous
import jax
import jax.numpy as jnp
from jax.experimental import pallas as pl
from jax.experimental.pallas import tpu as pltpu

_N = 100000
_D = 128
_K = 16
_S = 2
_BLK = 10000
_GRID = _N // (_S * _BLK)
_U = 25
_C = _BLK // (8 * _U)


def _chain_sumsq(x_ref, sel):
    y = x_ref[...].reshape(_U, _C * 8, _D)
    acc = None
    for c in range(_C):
        d = y[:, c * 8:(c + 1) * 8, :] - sel
        s = d * d
        acc = s if acc is None else acc + s
    return acc


def _body(gid_ref, sel_ref, *refs):
    x_refs, out_ref, acc_ref = refs[:_S], refs[_S], refs[_S + 1]
    i = pl.program_id(0)

    @pl.when(i == 0)
    def _init():
        acc_ref[...] = jnp.zeros_like(acc_ref)

    sub = gid_ref[0] % 8
    rows = jax.lax.broadcasted_iota(jnp.int32, (8, _D), 0)
    sel = jnp.sum(jnp.where(rows == sub, sel_ref[...], 0.0), axis=0,
                  keepdims=True)
    acc = None
    for r in x_refs:
        a = _chain_sumsq(r, sel)
        acc = a if acc is None else acc + a
    m = _U
    while m > 1:
        half = m // 2
        rest = acc[2 * half:]
        acc = acc[:half] + acc[half:2 * half]
        if rest.shape[0]:
            acc = jnp.concatenate([acc, rest], axis=0)
        m = acc.shape[0]
    acc_ref[...] += acc[0]

    @pl.when(i == _GRID - 1)
    def _finish():
        dist = jnp.sqrt(jnp.sum(acc_ref[...], axis=0, keepdims=True))
        lane = jax.lax.broadcasted_iota(jnp.int32, (1, _D), 1)
        total = jnp.float32(0.0)
        work = dist
        for _ in range(_K):
            mn = jnp.min(work)
            total = total + mn
            hit = work == mn
            first = jnp.min(jnp.where(hit, lane, _D))
            work = jnp.where(lane == first, jnp.float32(jnp.inf), work)
        out_ref[0, 0] = total / _K


def _stream_spec(s):
    return pl.BlockSpec((_BLK, _D), lambda i, g, s=s: (s * _GRID + i, 0))


@jax.jit
def _run(gid, x):
    grid_spec = pltpu.PrefetchScalarGridSpec(
        num_scalar_prefetch=1,
        grid=(_GRID,),
        in_specs=[pl.BlockSpec((8, _D), lambda i, g: (g[0] // 8, 0))]
        + [_stream_spec(s) for s in range(_S)],
        out_specs=pl.BlockSpec(memory_space=pltpu.SMEM),
        scratch_shapes=[pltpu.VMEM((8, _D), jnp.float32)],
    )
    out = pl.pallas_call(
        _body,
        grid_spec=grid_spec,
        out_shape=jax.ShapeDtypeStruct((1, 1), jnp.float32),
        compiler_params=pltpu.CompilerParams(
            dimension_semantics=("arbitrary",)),
    )(gid, x, *([x] * _S))
    return out[0, 0]


def kernel(group_id, all_latents):
    gid = jnp.asarray(group_id, jnp.int32).reshape(1)
    return _run(gid, all_latents)
